# baseline (speedup 1.0000x reference)
import jax
import jax.numpy as jnp
from jax.experimental import pallas as pl


def _body(x_ref, wt_ref, o_ref):
    o_ref[...] = jnp.dot(x_ref[...], wt_ref[...],
                         preferred_element_type=jnp.float32)


def kernel(x, W):
    T, D = x.shape
    E = W.shape[0]
    Wt = W.T
    BLK = 512
    grid = (T // BLK,)
    return pl.pallas_call(
        _body,
        grid=grid,
        in_specs=[
            pl.BlockSpec((BLK, D), lambda i: (i, 0)),
            pl.BlockSpec((D, E), lambda i: (0, 0)),
        ],
        out_specs=pl.BlockSpec((BLK, E), lambda i: (i, 0)),
        out_shape=jax.ShapeDtypeStruct((T, E), jnp.float32),
    )(x, Wt)


# speedup vs baseline: 1.7071x; 1.7071x over previous
import jax
import jax.numpy as jnp
from jax.experimental import pallas as pl


def _body(x_ref, wt_ref, o_ref):
    o_ref[...] = jnp.dot(x_ref[...], wt_ref[...],
                         preferred_element_type=jnp.float32)


def kernel(x, W):
    T, D = x.shape
    E = W.shape[0]
    Wt = W.T
    BLK = 4096
    grid = (T // BLK,)
    return pl.pallas_call(
        _body,
        grid=grid,
        in_specs=[
            pl.BlockSpec((BLK, D), lambda i: (i, 0)),
            pl.BlockSpec((D, E), lambda i: (0, 0)),
        ],
        out_specs=pl.BlockSpec((BLK, E), lambda i: (i, 0)),
        out_shape=jax.ShapeDtypeStruct((T, E), jnp.float32),
    )(x, Wt)
